# Initial kernel scaffold; baseline (speedup 1.0000x reference)
import jax
import jax.numpy as jnp
from jax.experimental import pallas as pl


def kernel(bank, weight_alloc):
    raise NotImplementedError("write your pallas kernel here")


# speedup vs baseline: 227.4237x; 227.4237x over previous
import jax
import jax.numpy as jnp
from jax import lax
from jax.experimental import pallas as pl
from jax.experimental.pallas import tpu as pltpu
from jax.experimental.pallas import tpu_sc as plsc

FULL_SHAPE = (100000, 128)
B = FULL_SHAPE[0] * FULL_SHAPE[1]
NC = 2
NS = 16
NW = NC * NS
PER_W = B // NW
CH = 16000
NITER = PER_W // CH


def _gather_body(bank_hbm, idx_hbm, out_hbm, idx_v, rows_v, sem):
    wid = lax.axis_index("s") * NC + lax.axis_index("c")
    base = wid * PER_W

    def step(i, carry):
        off = base + i * CH
        pltpu.sync_copy(idx_hbm.at[pl.ds(off, CH)], idx_v)
        pltpu.async_copy(bank_hbm.at[idx_v], rows_v, sem).wait()
        pltpu.sync_copy(rows_v, out_hbm.at[pl.ds(off, CH)])
        return carry

    lax.fori_loop(0, NITER, step, 0)


@jax.jit
def kernel(bank, weight_alloc):
    idx = weight_alloc.reshape(B).astype(jnp.int32)
    call = pl.kernel(
        _gather_body,
        out_type=jax.ShapeDtypeStruct((B,), jnp.float32),
        mesh=plsc.VectorSubcoreMesh(core_axis_name="c", subcore_axis_name="s"),
        scratch_types=[
            pltpu.VMEM((CH,), jnp.int32),
            pltpu.VMEM((CH,), jnp.float32),
            pltpu.SemaphoreType.DMA,
        ],
    )
    out = call(bank, idx)
    return out.reshape(FULL_SHAPE)


# speedup vs baseline: 839.0664x; 3.6894x over previous
import jax
import jax.numpy as jnp
from jax import lax
from jax.experimental import pallas as pl
from jax.experimental.pallas import tpu as pltpu
from jax.experimental.pallas import tpu_sc as plsc

FULL_SHAPE = (100000, 128)
B = FULL_SHAPE[0] * FULL_SHAPE[1]
NUM_W = 1280000
NC = 2
NS = 16
NW = NC * NS
PER_W = B // NW
CH = 10000
NITER = PER_W // CH
NPAIR = NITER // 2
STAGE = NUM_W // NS


def _gather_body(bank_hbm, idx_hbm, out_hbm, bank_sh,
                 idx0, idx1, rows0, rows1, sem_i, sem_g, sem_o):
    cid = lax.axis_index("c")
    sid = lax.axis_index("s")
    wid = sid * NC + cid
    base = wid * PER_W

    pltpu.sync_copy(bank_hbm.at[pl.ds(sid * STAGE, STAGE)],
                    bank_sh.at[pl.ds(sid * STAGE, STAGE)])
    pltpu.async_copy(idx_hbm.at[pl.ds(base, CH)], idx0, sem_i)
    plsc.subcore_barrier()

    def step(j, carry):
        off_a = base + (2 * j) * CH
        off_b = off_a + CH

        pltpu.make_async_copy(idx_hbm.at[pl.ds(off_a, CH)], idx0, sem_i).wait()
        pltpu.async_copy(idx_hbm.at[pl.ds(off_b, CH)], idx1, sem_i)

        @pl.when(j >= 1)
        def _():
            pltpu.make_async_copy(rows0, out_hbm.at[pl.ds(off_a, CH)],
                                  sem_o).wait()
        pltpu.async_copy(bank_sh.at[idx0], rows0, sem_g).wait()
        pltpu.async_copy(rows0, out_hbm.at[pl.ds(off_a, CH)], sem_o)

        pltpu.make_async_copy(idx_hbm.at[pl.ds(off_b, CH)], idx1, sem_i).wait()

        @pl.when(j + 1 < NPAIR)
        def _():
            pltpu.async_copy(idx_hbm.at[pl.ds(off_b + CH, CH)], idx0, sem_i)

        @pl.when(j >= 1)
        def _():
            pltpu.make_async_copy(rows1, out_hbm.at[pl.ds(off_b, CH)],
                                  sem_o).wait()
        pltpu.async_copy(bank_sh.at[idx1], rows1, sem_g).wait()
        pltpu.async_copy(rows1, out_hbm.at[pl.ds(off_b, CH)], sem_o)
        return carry

    lax.fori_loop(0, NPAIR, step, 0)
    pltpu.make_async_copy(rows0, out_hbm.at[pl.ds(base, CH)], sem_o).wait()
    pltpu.make_async_copy(rows1, out_hbm.at[pl.ds(base, CH)], sem_o).wait()


@jax.jit
def kernel(bank, weight_alloc):
    idx = weight_alloc.reshape(B).astype(jnp.int32)
    call = pl.kernel(
        _gather_body,
        out_type=jax.ShapeDtypeStruct((B,), jnp.float32),
        mesh=plsc.VectorSubcoreMesh(core_axis_name="c", subcore_axis_name="s"),
        scratch_types=[
            pltpu.VMEM_SHARED((NUM_W,), jnp.float32),
            pltpu.VMEM((CH,), jnp.int32),
            pltpu.VMEM((CH,), jnp.int32),
            pltpu.VMEM((CH,), jnp.float32),
            pltpu.VMEM((CH,), jnp.float32),
            pltpu.SemaphoreType.DMA,
            pltpu.SemaphoreType.DMA,
            pltpu.SemaphoreType.DMA,
        ],
    )
    out = call(bank, idx)
    return out.reshape(FULL_SHAPE)


# speedup vs baseline: 871.3135x; 1.0384x over previous
import jax
import jax.numpy as jnp
from jax import lax
from jax.experimental import pallas as pl
from jax.experimental.pallas import tpu as pltpu
from jax.experimental.pallas import tpu_sc as plsc

FULL_SHAPE = (100000, 128)
B = FULL_SHAPE[0] * FULL_SHAPE[1]
NUM_W = 1280000
NC = 2
NS = 16
NW = NC * NS
PER_W = B // NW
CH = 10000
NITER = PER_W // CH
NPAIR = NITER // 2
STAGE = NUM_W // NS


def _gather_body(bank_hbm, idx_hbm, out_hbm, bank_sh,
                 idx0, idx1, rows0, rows1, sem_i, sem_g, sem_o):
    cid = lax.axis_index("c")
    sid = lax.axis_index("s")
    wid = sid * NC + cid
    base = wid * PER_W

    pltpu.async_copy(idx_hbm.at[pl.ds(base, CH)], idx0, sem_i)
    pltpu.async_copy(idx_hbm.at[pl.ds(base + CH, CH)], idx1, sem_i)
    pltpu.sync_copy(bank_hbm.at[pl.ds(sid * STAGE, STAGE)],
                    bank_sh.at[pl.ds(sid * STAGE, STAGE)])
    plsc.subcore_barrier()
    pltpu.make_async_copy(idx_hbm.at[pl.ds(base, CH)], idx0, sem_i).wait()
    pltpu.async_copy(bank_sh.at[idx0], rows0, sem_g)

    def step(j, carry):
        off_a = base + (2 * j) * CH
        off_b = off_a + CH

        pltpu.make_async_copy(idx_hbm.at[pl.ds(off_b, CH)], idx1, sem_i).wait()

        @pl.when(j >= 1)
        def _():
            pltpu.make_async_copy(rows1, out_hbm.at[pl.ds(off_a, CH)],
                                  sem_o).wait()
        pltpu.async_copy(bank_sh.at[idx1], rows1, sem_g)
        pltpu.make_async_copy(bank_sh.at[idx0], rows0, sem_g).wait()
        pltpu.async_copy(rows0, out_hbm.at[pl.ds(off_a, CH)], sem_o)

        @pl.when(j + 1 < NPAIR)
        def _():
            pltpu.async_copy(idx_hbm.at[pl.ds(off_b + CH, CH)], idx0, sem_i)

        @pl.when(j + 1 < NPAIR)
        def _():
            pltpu.make_async_copy(idx_hbm.at[pl.ds(off_b + CH, CH)], idx0,
                                  sem_i).wait()
            pltpu.make_async_copy(rows0, out_hbm.at[pl.ds(off_a, CH)],
                                  sem_o).wait()
            pltpu.async_copy(bank_sh.at[idx0], rows0, sem_g)
        pltpu.make_async_copy(bank_sh.at[idx1], rows1, sem_g).wait()
        pltpu.async_copy(rows1, out_hbm.at[pl.ds(off_b, CH)], sem_o)

        @pl.when(j + 1 < NPAIR)
        def _():
            pltpu.async_copy(idx_hbm.at[pl.ds(off_b + 2 * CH, CH)], idx1,
                             sem_i)
        return carry

    lax.fori_loop(0, NPAIR, step, 0)
    pltpu.make_async_copy(rows0, out_hbm.at[pl.ds(base, CH)], sem_o).wait()
    pltpu.make_async_copy(rows1, out_hbm.at[pl.ds(base, CH)], sem_o).wait()


@jax.jit
def kernel(bank, weight_alloc):
    idx = weight_alloc.reshape(B).astype(jnp.int32)
    call = pl.kernel(
        _gather_body,
        out_type=jax.ShapeDtypeStruct((B,), jnp.float32),
        mesh=plsc.VectorSubcoreMesh(core_axis_name="c", subcore_axis_name="s"),
        scratch_types=[
            pltpu.VMEM_SHARED((NUM_W,), jnp.float32),
            pltpu.VMEM((CH,), jnp.int32),
            pltpu.VMEM((CH,), jnp.int32),
            pltpu.VMEM((CH,), jnp.float32),
            pltpu.VMEM((CH,), jnp.float32),
            pltpu.SemaphoreType.DMA,
            pltpu.SemaphoreType.DMA,
            pltpu.SemaphoreType.DMA,
        ],
    )
    out = call(bank, idx)
    return out.reshape(FULL_SHAPE)
